# baseline (speedup 1.0000x reference)
import functools

import jax
import jax.numpy as jnp
from jax import lax
from jax.experimental import pallas as pl
from jax.experimental.pallas import tpu as pltpu
from jax.experimental.pallas import tpu_sc as plsc

D_MODEL = 128
MAX_LEN = 366
NUM_CORES = 2
NUM_SUBCORES = 16
NUM_WORKERS = NUM_CORES * NUM_SUBCORES
GROUP = 128
K = 2
NBUF = 2


def _body(n_groups, pe_hbm, x_hbm, out_hbm, table_v, idx_v, bufs, sem, ssems):
    wid = lax.axis_index("s") * NUM_CORES + lax.axis_index("c")
    row0 = wid * n_groups

    @pl.when(lax.axis_index("s") == 0)
    def _():
        pltpu.sync_copy(pe_hbm, table_v)

    plsc.subcore_barrier()
    pltpu.sync_copy(x_hbm.at[pl.ds(row0, n_groups)], idx_v)

    def clip_group(g):
        for c in range(0, GROUP, 16):
            v = idx_v[g, pl.ds(c, 16)]
            idx_v[g, pl.ds(c, 16)] = lax.max(lax.min(v, MAX_LEN), 1) - 1

    def fire_gathers(sg, b):
        for j in range(K):
            pltpu.async_copy(
                table_v.at[idx_v.at[sg * K + j]],
                bufs[b].at[pl.ds(j * GROUP, GROUP)],
                sem,
            )

    def wait_gathers(sg, b):
        for j in range(K):
            pltpu.make_async_copy(
                table_v.at[idx_v.at[sg * K + j]],
                bufs[b].at[pl.ds(j * GROUP, GROUP)],
                sem,
            ).wait()

    n_super = n_groups // K

    for b in range(NBUF):
        for j in range(K):
            clip_group(b * K + j)
        fire_gathers(b, b)

    def ring_body(i0, _):
        for b in range(NBUF):
            sg = i0 * NBUF + b
            sgn = sg + NBUF
            wait_gathers(sg, b)
            scat = pltpu.async_copy(
                bufs[b],
                out_hbm.at[pl.ds((row0 + sg * K) * GROUP, K * GROUP)],
                ssems[b],
            )

            @pl.when(sgn < n_super)
            def _():
                for j in range(K):
                    clip_group(sgn * K + j)
                scat.wait()
                fire_gathers(sgn, b)

            @pl.when(sgn >= n_super)
            def _():
                scat.wait()

        return 0

    lax.fori_loop(0, n_super // NBUF, ring_body, 0)


@functools.partial(jax.jit, static_argnames=())
def kernel(x, pe):
    b, h = x.shape
    n = b * h
    assert n % (NUM_WORKERS * GROUP) == 0
    n_groups = n // (NUM_WORKERS * GROUP)
    assert n_groups % (K * NBUF) == 0
    x2d = x.reshape(n // GROUP, GROUP)

    mesh = plsc.VectorSubcoreMesh(core_axis_name="c", subcore_axis_name="s")
    run = pl.kernel(
        functools.partial(_body, n_groups),
        mesh=mesh,
        out_type=jax.ShapeDtypeStruct((n, D_MODEL), jnp.float32),
        scratch_types=[
            pltpu.VMEM_SHARED((MAX_LEN, D_MODEL), jnp.float32),
            pltpu.VMEM((n_groups, GROUP), jnp.int32),
            [pltpu.VMEM((K * GROUP, D_MODEL), jnp.float32) for _ in range(NBUF)],
            pltpu.SemaphoreType.DMA,
            [pltpu.SemaphoreType.DMA for _ in range(NBUF)],
        ],
    )
    out = run(pe, x2d)
    return out.reshape(b, h, D_MODEL)


# speedup vs baseline: 1.0075x; 1.0075x over previous
import functools

import jax
import jax.numpy as jnp
from jax import lax
from jax.experimental import pallas as pl
from jax.experimental.pallas import tpu as pltpu
from jax.experimental.pallas import tpu_sc as plsc

D_MODEL = 128
MAX_LEN = 366
NUM_CORES = 2
NUM_SUBCORES = 16
NUM_WORKERS = NUM_CORES * NUM_SUBCORES
GROUP = 128
K = 1
NBUF = 5


def _body(n_groups, pe_hbm, x_hbm, out_hbm, table_v, idx_v, bufs, sem, ssems):
    wid = lax.axis_index("s") * NUM_CORES + lax.axis_index("c")
    row0 = wid * n_groups

    sid = lax.axis_index("s")

    @pl.when(sid == 0)
    def _():
        pltpu.sync_copy(pe_hbm, table_v)

    plsc.subcore_barrier()
    pltpu.sync_copy(x_hbm.at[pl.ds(row0, n_groups)], idx_v)

    def clip_group(g):
        for c in range(0, GROUP, 16):
            v = idx_v[g, pl.ds(c, 16)]
            idx_v[g, pl.ds(c, 16)] = lax.max(lax.min(v, MAX_LEN), 1) - 1

    def fire_gathers(sg, b):
        for j in range(K):
            pltpu.async_copy(
                table_v.at[idx_v.at[sg * K + j]],
                bufs[b].at[pl.ds(j * GROUP, GROUP)],
                sem,
            )

    def wait_gathers(sg, b):
        for j in range(K):
            pltpu.make_async_copy(
                table_v.at[idx_v.at[sg * K + j]],
                bufs[b].at[pl.ds(j * GROUP, GROUP)],
                sem,
            ).wait()

    n_super = n_groups // K

    for b in range(NBUF):
        for j in range(K):
            clip_group(b * K + j)
        fire_gathers(b, b)

    def ring_body(i0, _):
        for b in range(NBUF):
            sg = i0 * NBUF + b
            sgn = sg + NBUF
            wait_gathers(sg, b)
            scat = pltpu.async_copy(
                bufs[b],
                out_hbm.at[pl.ds((row0 + sg * K) * GROUP, K * GROUP)],
                ssems[b],
            )

            @pl.when(sgn < n_super)
            def _():
                for j in range(K):
                    clip_group(sgn * K + j)
                scat.wait()
                fire_gathers(sgn, b)

            @pl.when(sgn >= n_super)
            def _():
                scat.wait()

        return 0

    lax.fori_loop(0, n_super // NBUF, ring_body, 0)


@functools.partial(jax.jit, static_argnames=())
def kernel(x, pe):
    b, h = x.shape
    n = b * h
    assert n % (NUM_WORKERS * GROUP) == 0
    n_groups = n // (NUM_WORKERS * GROUP)
    assert n_groups % (K * NBUF) == 0
    x2d = x.reshape(n // GROUP, GROUP)

    mesh = plsc.VectorSubcoreMesh(core_axis_name="c", subcore_axis_name="s")
    run = pl.kernel(
        functools.partial(_body, n_groups),
        mesh=mesh,
        out_type=jax.ShapeDtypeStruct((n, D_MODEL), jnp.float32),
        scratch_types=[
            pltpu.VMEM_SHARED((MAX_LEN, D_MODEL), jnp.float32),
            pltpu.VMEM((n_groups, GROUP), jnp.int32),
            [pltpu.VMEM((K * GROUP, D_MODEL), jnp.float32) for _ in range(NBUF)],
            pltpu.SemaphoreType.DMA,
            [pltpu.SemaphoreType.DMA for _ in range(NBUF)],
        ],
    )
    out = run(pe, x2d)
    return out.reshape(b, h, D_MODEL)


# speedup vs baseline: 1.0080x; 1.0005x over previous
import functools

import jax
import jax.numpy as jnp
from jax import lax
from jax.experimental import pallas as pl
from jax.experimental.pallas import tpu as pltpu
from jax.experimental.pallas import tpu_sc as plsc

D_MODEL = 128
MAX_LEN = 366
NUM_CORES = 2
NUM_SUBCORES = 16
NUM_WORKERS = NUM_CORES * NUM_SUBCORES
GROUP = 128
K = 1
NBUF = 5


def _body(n_groups, pe_hbm, x_hbm, out_hbm, table_v, idx_v, bufs, sem, ssems):
    wid = lax.axis_index("s") * NUM_CORES + lax.axis_index("c")
    row0 = wid * n_groups

    sid = lax.axis_index("s")

    @pl.when(sid == 0)
    def _():
        pltpu.sync_copy(pe_hbm, table_v)

    pltpu.sync_copy(x_hbm.at[pl.ds(row0, n_groups)], idx_v)

    def clip_group(g):
        for c in range(0, GROUP, 16):
            v = idx_v[g, pl.ds(c, 16)]
            idx_v[g, pl.ds(c, 16)] = lax.max(lax.min(v, MAX_LEN), 1) - 1

    def fire_gathers(sg, b):
        for j in range(K):
            pltpu.async_copy(
                table_v.at[idx_v.at[sg * K + j]],
                bufs[b].at[pl.ds(j * GROUP, GROUP)],
                sem,
            )

    def wait_gathers(sg, b):
        for j in range(K):
            pltpu.make_async_copy(
                table_v.at[idx_v.at[sg * K + j]],
                bufs[b].at[pl.ds(j * GROUP, GROUP)],
                sem,
            ).wait()

    n_super = n_groups // K

    for b in range(NBUF):
        for j in range(K):
            clip_group(b * K + j)
    plsc.subcore_barrier()
    for b in range(NBUF):
        fire_gathers(b, b)

    def ring_body(i0, _):
        for b in range(NBUF):
            sg = i0 * NBUF + b
            sgn = sg + NBUF
            wait_gathers(sg, b)
            scat = pltpu.async_copy(
                bufs[b],
                out_hbm.at[pl.ds((row0 + sg * K) * GROUP, K * GROUP)],
                ssems[b],
            )

            @pl.when(sgn < n_super)
            def _():
                for j in range(K):
                    clip_group(sgn * K + j)
                scat.wait()
                fire_gathers(sgn, b)

            @pl.when(sgn >= n_super)
            def _():
                scat.wait()

        return 0

    lax.fori_loop(0, n_super // NBUF, ring_body, 0)


@functools.partial(jax.jit, static_argnames=())
def kernel(x, pe):
    b, h = x.shape
    n = b * h
    assert n % (NUM_WORKERS * GROUP) == 0
    n_groups = n // (NUM_WORKERS * GROUP)
    assert n_groups % (K * NBUF) == 0
    x2d = x.reshape(n // GROUP, GROUP)

    mesh = plsc.VectorSubcoreMesh(core_axis_name="c", subcore_axis_name="s")
    run = pl.kernel(
        functools.partial(_body, n_groups),
        mesh=mesh,
        out_type=jax.ShapeDtypeStruct((n, D_MODEL), jnp.float32),
        scratch_types=[
            pltpu.VMEM_SHARED((MAX_LEN, D_MODEL), jnp.float32),
            pltpu.VMEM((n_groups, GROUP), jnp.int32),
            [pltpu.VMEM((K * GROUP, D_MODEL), jnp.float32) for _ in range(NBUF)],
            pltpu.SemaphoreType.DMA,
            [pltpu.SemaphoreType.DMA for _ in range(NBUF)],
        ],
    )
    out = run(pe, x2d)
    return out.reshape(b, h, D_MODEL)
